# baseline (speedup 1.0000x reference)
import functools

import jax
import jax.numpy as jnp
from jax import lax
from jax.experimental import pallas as pl
from jax.experimental.pallas import tpu as pltpu
from jax.experimental.pallas import tpu_sc as plsc

_RATIOS = (0.317, 0.329, 0.326, 0.115, 0.701, 0.367, 1.22, 0.241)
_SMOOTH = 0.0001

_B = 32
_N = 32768
_L = 16
_NV = _N // _L

_IMIN = jnp.int32(-2147483648)
_NB1 = 4096
_NB2 = 1024
_NB3 = 1024
_HWORDS = _NB1 + _NB2 + _NB3
_FBASE = 0


def _sc_body(inp_hbm, tgt_hbm, label_hbm, out_hbm,
             inp_v, tgt_v, label_v, keys_v, hist_v, fhist_v, out_v,
             sem1, sem2, sem3):
    wid = lax.axis_index("s") * 2 + lax.axis_index("c")
    h1 = pltpu.async_copy(inp_hbm.at[wid], inp_v, sem1)
    h2 = pltpu.async_copy(tgt_hbm.at[wid], tgt_v, sem2)
    h3 = pltpu.async_copy(label_hbm, label_v, sem3)

    zeros = jnp.zeros((_L,), jnp.int32)
    fzeros = jnp.zeros((_L,), jnp.float32)
    ones = jnp.ones((_L,), jnp.int32)
    lane = lax.iota(jnp.int32, _L)

    @plsc.parallel_loop(0, _HWORDS // _L, unroll=2)
    def _(i):
        off = pl.multiple_of(i * _L, _L)
        hist_v[pl.ds(off, _L)] = zeros

    @plsc.parallel_loop(0, _NB3 // _L, unroll=2)
    def _(i):
        fhist_v[pl.ds(pl.multiple_of(i * _L, _L), _L)] = fzeros

    h3.wait()

    l0 = label_v[pl.ds(0, _L)]
    l1 = label_v[pl.ds(_L, _L)]
    lab_vec = jnp.where((wid < _L) & (lane == wid), l0, 0) + \
        jnp.where((wid >= _L) & (lane == wid - _L), l1, 0)
    mylab = jnp.sum(lab_vec)
    ratio = jnp.float32(_RATIOS[0])
    for j in range(1, 8):
        ratio = jnp.where(mylab == j, jnp.float32(_RATIOS[j]), ratio)

    h1.wait()
    h2.wait()

    @plsc.parallel_loop(0, _NV, unroll=2, carry=(zeros, fzeros))
    def acc(i, c):
        acc_t, acc_pf = c
        off = pl.multiple_of(i * _L, _L)
        x = inp_v[pl.ds(off, _L)]
        t = tgt_v[pl.ds(off, _L)]
        b = plsc.bitcast(x, jnp.int32)
        sgn = lax.shift_right_arithmetic(b, jnp.int32(31))
        key = b ^ (sgn | _IMIN)
        key = jnp.where(t > 0, jnp.int32(-1), key)
        keys_v[pl.ds(off, _L)] = key
        s = 1.0 / (1.0 + jnp.exp(-x))
        q = 1.0 - s
        fi = q * q * s
        inp_v[pl.ds(off, _L)] = fi
        d1 = lax.shift_right_logical(key, jnp.int32(20))
        plsc.addupdate_scatter(hist_v, [d1], ones)
        return acc_t + t, acc_pf + jnp.where(t > 0, fi, 0.0)

    acc_t, acc_pf = acc
    pos_num = jnp.sum(acc_t)
    posfi = jnp.sum(acc_pf)

    neg_num = _N - pos_num
    keep = jnp.minimum((pos_num.astype(jnp.float32) * ratio).astype(jnp.int32),
                       neg_num)
    k_idx = jnp.where(keep > 1, neg_num - keep + 1, 1 - keep)

    def scan_hist(base, nbins, k_rem):
        def body(i, carry):
            nb, cb, run = carry
            off = base + pl.multiple_of(i * _L, _L)
            h = hist_v[pl.ds(off, _L)]
            cum = jnp.cumsum(h) + run
            mle = cum <= k_rem
            nb = nb + jnp.where(mle, 1, 0)
            cb = cb + jnp.where(mle, h, 0)
            run = run + jnp.broadcast_to(jnp.sum(h), (_L,))
            return nb, cb, run

        nb, cb, _ = lax.fori_loop(0, nbins // _L, body, (zeros, zeros, zeros))
        return jnp.sum(nb), jnp.sum(cb)

    t1, cb1 = scan_hist(0, _NB1, k_idx)
    k2 = k_idx - cb1

    @plsc.parallel_loop(0, _NV, unroll=2)
    def _(i):
        off = pl.multiple_of(i * _L, _L)
        k = keys_v[pl.ds(off, _L)]
        m = lax.shift_right_logical(k, jnp.int32(20)) == t1
        d = lax.shift_right_logical(k, jnp.int32(10)) & jnp.int32(_NB2 - 1)
        plsc.addupdate_scatter(hist_v, [d + jnp.int32(_NB1)], ones, mask=m)

    t2, cb2 = scan_hist(_NB1, _NB2, k2)
    k3 = k2 - cb2
    p2 = (t1 << 10) | t2

    @plsc.parallel_loop(0, _NV, unroll=2, carry=fzeros)
    def acc_nf(i, a):
        off = pl.multiple_of(i * _L, _L)
        k = keys_v[pl.ds(off, _L)]
        fi = inp_v[pl.ds(off, _L)]
        hi = lax.shift_right_logical(k, jnp.int32(10))
        fin = k != -1
        m = hi == p2
        d = k & jnp.int32(_NB3 - 1)
        plsc.addupdate_scatter(hist_v, [d + jnp.int32(_NB1 + _NB2)], ones,
                               mask=m)
        plsc.addupdate_scatter(fhist_v, [d], fi, mask=m & fin)
        return a + jnp.where((hi > p2) & fin, fi, 0.0)

    negfi_coarse = jnp.sum(acc_nf)

    def fscan_body(i, carry):
        nb, fb, ft, cnt = carry
        off = pl.multiple_of(i * _L, _L)
        hf = fhist_v[pl.ds(off, _L)]
        h = hist_v[pl.ds(_NB1 + _NB2 + off, _L)]
        cum = jnp.cumsum(h) + cnt
        mle = cum <= k3
        nb = nb + jnp.where(mle, 1, 0)
        fb = fb + jnp.where(mle, hf, 0.0)
        ft = ft + hf
        cnt = cnt + jnp.broadcast_to(jnp.sum(h), (_L,))
        return nb, fb, ft, cnt

    nb3v, fb3v, ft3v, _ = lax.fori_loop(
        0, _NB3 // _L, fscan_body, (zeros, fzeros, fzeros, zeros))
    t3 = jnp.sum(nb3v)
    fb3 = jnp.sum(fb3v)
    ft3 = jnp.sum(ft3v)

    voff = (t3 >> 4) << 4
    fvec = fhist_v[pl.ds(voff, _L)]
    f3sel = jnp.sum(jnp.where(lane == (t3 & 15), fvec, 0.0))

    negfi = negfi_coarse + (ft3 - fb3 - f3sel)

    res = jnp.where(lane == 0, jnp.broadcast_to(posfi, (_L,)),
                    jnp.where(lane == 1, jnp.broadcast_to(negfi, (_L,)),
                              jnp.where(lane == 2,
                                        jnp.broadcast_to(
                                            pos_num.astype(jnp.float32), (_L,)),
                                        0.0)))
    out_v[...] = res
    pltpu.sync_copy(out_v, out_hbm.at[wid])


def _sc_rowstats(inp, tgt, label):
    call = functools.partial(
        pl.kernel,
        out_type=jax.ShapeDtypeStruct((_B, _L), jnp.float32),
        mesh=plsc.VectorSubcoreMesh(core_axis_name="c", subcore_axis_name="s"),
        compiler_params=pltpu.CompilerParams(needs_layout_passes=False),
        scratch_types=[
            pltpu.VMEM((_N,), jnp.float32),
            pltpu.VMEM((_N,), jnp.int32),
            pltpu.VMEM((_B,), jnp.int32),
            pltpu.VMEM((_N,), jnp.int32),
            pltpu.VMEM((_HWORDS,), jnp.int32),
            pltpu.VMEM((_NB3,), jnp.float32),
            pltpu.VMEM((_L,), jnp.float32),
            pltpu.SemaphoreType.DMA,
            pltpu.SemaphoreType.DMA,
            pltpu.SemaphoreType.DMA,
        ],
    )(_sc_body)
    return call(inp, tgt, label)


def _tc_combine(pr_ref, out_ref):
    pr = pr_ref[...]
    inter = pr[:, 0:1]
    negfi = pr[:, 1:2]
    posn = pr[:, 2:3]
    denom = jnp.sum(inter) + jnp.sum(negfi) + jnp.sum(posn)
    out_ref[...] = 1.0 - (2.0 * inter + _SMOOTH) / (denom + _SMOOTH)


def kernel(input, target, label):
    perrow = _sc_rowstats(input, target, label)
    loss = pl.pallas_call(
        _tc_combine,
        out_shape=jax.ShapeDtypeStruct((_B, 1), jnp.float32),
    )(perrow)
    return loss[:, 0]


# speedup vs baseline: 1.0465x; 1.0465x over previous
import functools

import jax
import jax.numpy as jnp
from jax import lax
from jax.experimental import pallas as pl
from jax.experimental.pallas import tpu as pltpu
from jax.experimental.pallas import tpu_sc as plsc

_RATIOS = (0.317, 0.329, 0.326, 0.115, 0.701, 0.367, 1.22, 0.241)
_SMOOTH = 0.0001

_B = 32
_N = 32768
_L = 16
_NV = _N // _L

_IMIN = jnp.int32(-2147483648)
_NB1 = 4096
_NB2 = 1024
_NB3 = 1024
_HWORDS = _NB1 + _NB2 + _NB3
_FBASE = 0


_NCH = 4
_CHW = _N // _NCH
_CHV = _NV // _NCH


def _sc_body(inp_hbm, tgt_hbm, label_hbm, out_hbm,
             inp_v, tgt_v, label_v, keys_v, hist_v, fhist_v, out_v,
             semi0, semi1, semt0, semt1, sem3):
    wid = lax.axis_index("s") * 2 + lax.axis_index("c")
    semi = (semi0, semi1)
    semt = (semt0, semt1)

    def start_chunk(c):
        sl = pl.ds(c * _CHW, _CHW)
        hi = pltpu.async_copy(inp_hbm.at[wid, sl], inp_v.at[sl], semi[c % 2])
        ht = pltpu.async_copy(tgt_hbm.at[wid, sl], tgt_v.at[sl], semt[c % 2])
        return hi, ht

    pend = start_chunk(0)
    h3 = pltpu.async_copy(label_hbm, label_v, sem3)

    zeros = jnp.zeros((_L,), jnp.int32)
    fzeros = jnp.zeros((_L,), jnp.float32)
    ones = jnp.ones((_L,), jnp.int32)
    lane = lax.iota(jnp.int32, _L)

    @plsc.parallel_loop(0, _HWORDS // _L, unroll=4)
    def _(i):
        off = pl.multiple_of(i * _L, _L)
        hist_v[pl.ds(off, _L)] = zeros

    @plsc.parallel_loop(0, _NB3 // _L, unroll=4)
    def _(i):
        fhist_v[pl.ds(pl.multiple_of(i * _L, _L), _L)] = fzeros

    h3.wait()

    l0 = label_v[pl.ds(0, _L)]
    l1 = label_v[pl.ds(_L, _L)]
    lab_vec = jnp.where((wid < _L) & (lane == wid), l0, 0) + \
        jnp.where((wid >= _L) & (lane == wid - _L), l1, 0)
    mylab = jnp.sum(lab_vec)
    ratio = jnp.float32(_RATIOS[0])
    for j in range(1, 8):
        ratio = jnp.where(mylab == j, jnp.float32(_RATIOS[j]), ratio)

    def p1_body(i, c):
        acc_t, acc_pf = c
        off = pl.multiple_of(i * _L, _L)
        x = inp_v[pl.ds(off, _L)]
        t = tgt_v[pl.ds(off, _L)]
        b = plsc.bitcast(x, jnp.int32)
        sgn = lax.shift_right_arithmetic(b, jnp.int32(31))
        key = b ^ (sgn | _IMIN)
        key = jnp.where(t > 0, jnp.int32(-1), key)
        keys_v[pl.ds(off, _L)] = key
        s = 1.0 / (1.0 + jnp.exp(-x))
        q = 1.0 - s
        fi = q * q * s
        inp_v[pl.ds(off, _L)] = fi
        d1 = lax.shift_right_logical(key, jnp.int32(20))
        plsc.addupdate_scatter(hist_v, [d1], ones)
        return acc_t + t, acc_pf + jnp.where(t > 0, fi, 0.0)

    carry = (zeros, fzeros)
    for ch in range(_NCH):
        hi, ht = pend
        if ch + 1 < _NCH:
            nxt = start_chunk(ch + 1)
        hi.wait()
        ht.wait()
        carry = plsc.parallel_loop(
            ch * _CHV, (ch + 1) * _CHV, unroll=4, carry=carry)(p1_body)
        if ch + 1 < _NCH:
            pend = nxt

    acc_t, acc_pf = carry
    pos_num = jnp.sum(acc_t)
    posfi = jnp.sum(acc_pf)

    neg_num = _N - pos_num
    keep = jnp.minimum((pos_num.astype(jnp.float32) * ratio).astype(jnp.int32),
                       neg_num)
    k_idx = jnp.where(keep > 1, neg_num - keep + 1, 1 - keep)

    def scan_hist(base, nbins, k_rem):
        def body(i, carry):
            nb, cb, run = carry
            off = base + pl.multiple_of(i * _L, _L)
            h = hist_v[pl.ds(off, _L)]
            cum = jnp.cumsum(h) + run
            mle = cum <= k_rem
            nb = nb + jnp.where(mle, 1, 0)
            cb = cb + jnp.where(mle, h, 0)
            run = run + jnp.broadcast_to(jnp.sum(h), (_L,))
            return nb, cb, run

        nb, cb, _ = lax.fori_loop(0, nbins // _L, body, (zeros, zeros, zeros))
        return jnp.sum(nb), jnp.sum(cb)

    t1, cb1 = scan_hist(0, _NB1, k_idx)
    k2 = k_idx - cb1

    @plsc.parallel_loop(0, _NV, unroll=4)
    def _(i):
        off = pl.multiple_of(i * _L, _L)
        k = keys_v[pl.ds(off, _L)]
        m = lax.shift_right_logical(k, jnp.int32(20)) == t1
        d = lax.shift_right_logical(k, jnp.int32(10)) & jnp.int32(_NB2 - 1)
        plsc.addupdate_scatter(hist_v, [d + jnp.int32(_NB1)], ones, mask=m)

    t2, cb2 = scan_hist(_NB1, _NB2, k2)
    k3 = k2 - cb2
    p2 = (t1 << 10) | t2

    @plsc.parallel_loop(0, _NV, unroll=4, carry=fzeros)
    def acc_nf(i, a):
        off = pl.multiple_of(i * _L, _L)
        k = keys_v[pl.ds(off, _L)]
        fi = inp_v[pl.ds(off, _L)]
        hi = lax.shift_right_logical(k, jnp.int32(10))
        fin = k != -1
        m = hi == p2
        d = k & jnp.int32(_NB3 - 1)
        plsc.addupdate_scatter(hist_v, [d + jnp.int32(_NB1 + _NB2)], ones,
                               mask=m)
        plsc.addupdate_scatter(fhist_v, [d], fi, mask=m & fin)
        return a + jnp.where((hi > p2) & fin, fi, 0.0)

    negfi_coarse = jnp.sum(acc_nf)

    def fscan_body(i, carry):
        nb, fb, ft, cnt = carry
        off = pl.multiple_of(i * _L, _L)
        hf = fhist_v[pl.ds(off, _L)]
        h = hist_v[pl.ds(_NB1 + _NB2 + off, _L)]
        cum = jnp.cumsum(h) + cnt
        mle = cum <= k3
        nb = nb + jnp.where(mle, 1, 0)
        fb = fb + jnp.where(mle, hf, 0.0)
        ft = ft + hf
        cnt = cnt + jnp.broadcast_to(jnp.sum(h), (_L,))
        return nb, fb, ft, cnt

    nb3v, fb3v, ft3v, _ = lax.fori_loop(
        0, _NB3 // _L, fscan_body, (zeros, fzeros, fzeros, zeros))
    t3 = jnp.sum(nb3v)
    fb3 = jnp.sum(fb3v)
    ft3 = jnp.sum(ft3v)

    voff = (t3 >> 4) << 4
    fvec = fhist_v[pl.ds(voff, _L)]
    f3sel = jnp.sum(jnp.where(lane == (t3 & 15), fvec, 0.0))

    negfi = negfi_coarse + (ft3 - fb3 - f3sel)

    res = jnp.where(lane == 0, jnp.broadcast_to(posfi, (_L,)),
                    jnp.where(lane == 1, jnp.broadcast_to(negfi, (_L,)),
                              jnp.where(lane == 2,
                                        jnp.broadcast_to(
                                            pos_num.astype(jnp.float32), (_L,)),
                                        0.0)))
    out_v[...] = res
    pltpu.sync_copy(out_v, out_hbm.at[wid])


def _sc_rowstats(inp, tgt, label):
    call = functools.partial(
        pl.kernel,
        out_type=jax.ShapeDtypeStruct((_B, _L), jnp.float32),
        mesh=plsc.VectorSubcoreMesh(core_axis_name="c", subcore_axis_name="s"),
        compiler_params=pltpu.CompilerParams(needs_layout_passes=False),
        scratch_types=[
            pltpu.VMEM((_N,), jnp.float32),
            pltpu.VMEM((_N,), jnp.int32),
            pltpu.VMEM((_B,), jnp.int32),
            pltpu.VMEM((_N,), jnp.int32),
            pltpu.VMEM((_HWORDS,), jnp.int32),
            pltpu.VMEM((_NB3,), jnp.float32),
            pltpu.VMEM((_L,), jnp.float32),
            pltpu.SemaphoreType.DMA,
            pltpu.SemaphoreType.DMA,
            pltpu.SemaphoreType.DMA,
            pltpu.SemaphoreType.DMA,
            pltpu.SemaphoreType.DMA,
        ],
    )(_sc_body)
    return call(inp, tgt, label)


def _tc_combine(pr_ref, out_ref):
    pr = pr_ref[...]
    inter = pr[:, 0:1]
    negfi = pr[:, 1:2]
    posn = pr[:, 2:3]
    denom = jnp.sum(inter) + jnp.sum(negfi) + jnp.sum(posn)
    out_ref[...] = 1.0 - (2.0 * inter + _SMOOTH) / (denom + _SMOOTH)


def kernel(input, target, label):
    perrow = _sc_rowstats(input, target, label)
    loss = pl.pallas_call(
        _tc_combine,
        out_shape=jax.ShapeDtypeStruct((_B, 1), jnp.float32),
    )(perrow)
    return loss[:, 0]
